# baseline (speedup 1.0000x reference)
import functools

import jax
import jax.numpy as jnp
from jax import lax
from jax.experimental import pallas as pl
from jax.experimental.pallas import tpu as pltpu
from jax.experimental.pallas import tpu_sc as plsc

_NUM_CORES = 2
_NUM_SUBCORES = 16
_NUM_WORKERS = _NUM_CORES * _NUM_SUBCORES

_BLK = 128
_GRP = 4
_PITCH = _BLK + 1


@functools.lru_cache(maxsize=None)
def _make_gather(n_b: int, n_s: int, d: int):
    assert d == 64 and n_b % _BLK == 0
    n_bh = n_b // _BLK
    n_blocks = n_s * n_bh
    blocks_per_w = n_blocks // _NUM_WORKERS
    groups_per_w = blocks_per_w // _GRP
    assert blocks_per_w % _GRP == 0 and groups_per_w % 2 == 0

    mesh = plsc.VectorSubcoreMesh(
        core_axis_name="c", subcore_axis_name="s",
        num_cores=_NUM_CORES, num_subcores=_NUM_SUBCORES)

    @functools.partial(
        pl.kernel,
        out_type=jax.ShapeDtypeStruct((n_s, 8, n_bh, 8, _BLK), jnp.float32),
        mesh=mesh,
        scratch_types=[
            [pltpu.VMEM((_GRP, _BLK), jnp.int32)] * 4,
            [pltpu.VMEM((_BLK, d), jnp.float32)] * _GRP,
            [pltpu.VMEM((8, _GRP, 8, _PITCH), jnp.float32)] * 2,
            [pltpu.SemaphoreType.DMA] * 4,
            [pltpu.SemaphoreType.DMA] * _GRP,
            [pltpu.SemaphoreType.DMA] * 2,
        ],
        compiler_params=pltpu.CompilerParams(
            use_tc_tiling_on_sc=False, needs_layout_passes=False),
    )
    def gather_kernel(table, idx_hbm, out5,
                      ibufs, gbufs, tbufs, isems, gsems, ssems):
        wid = lax.axis_index("s") * _NUM_CORES + lax.axis_index("c")
        blk_base = wid * blocks_per_w

        d16 = lax.iota(jnp.int32, 16)
        dh_vecs = [(d16 + 16 * k) >> 3 for k in range(4)]
        dl_vecs = [(d16 + 16 * k) & 7 for k in range(4)]
        r_vecs = [jnp.full((16,), r, jnp.int32) for r in range(_GRP)]

        def fire_idx(g, i):
            pltpu.async_copy(
                idx_hbm.at[pl.ds(blk_base + _GRP * g, _GRP)],
                ibufs[i], isems[i])

        def wait_idx(i):
            pltpu.make_async_copy(
                idx_hbm.at[pl.ds(0, _GRP)], ibufs[i], isems[i]).wait()

        def fire_gather(g, i, r):
            pltpu.async_copy(
                table.at[ibufs[i].at[r]], gbufs[r], gsems[r])

        def wait_gather(r):
            pltpu.make_async_copy(
                table.at[ibufs[0].at[0]], gbufs[r], gsems[r]).wait()

        def fire_store(g, tb):
            kbase = blk_base + _GRP * g
            s = kbase >> 5
            bh0 = kbase & (n_bh - 1)
            pltpu.async_copy(tbufs[tb].at[:, :, :, pl.ds(0, _BLK)],
                             out5.at[s, :, pl.ds(bh0, _GRP)], ssems[tb])

        def wait_store(tb):
            pltpu.make_async_copy(
                tbufs[tb].at[:, :, :, pl.ds(0, _BLK)],
                out5.at[0, :, pl.ds(0, _GRP)], ssems[tb]).wait()

        def transpose_block(r, tb):
            buf_r, tbuf_r = gbufs[r], tbufs[tb]

            @pl.loop(0, _BLK, unroll=8)
            def _(bl):
                blv = jnp.full((16,), bl, jnp.int32)
                vs = [buf_r[bl, pl.ds(16 * k, 16)] for k in range(4)]
                for k in range(4):
                    plsc.store_scatter(
                        tbuf_r, [dh_vecs[k], r_vecs[r], dl_vecs[k], blv],
                        vs[k])

        def step(g, tb, ip1, guarded):
            if guarded:
                wait_store(tb)

                @pl.when(g + 1 < groups_per_w)
                def _():
                    wait_idx(ip1)

                @pl.when(g + 3 < groups_per_w)
                def _():
                    fire_idx(g + 3, (ip1 + 2) % 4)
            else:
                wait_idx(ip1)
                fire_idx(g + 3, (ip1 + 2) % 4)
            for r in range(_GRP):
                wait_gather(r)
                transpose_block(r, tb)
                if guarded:

                    @pl.when(g + 1 < groups_per_w)
                    def _():
                        fire_gather(g + 1, ip1, r)
                else:
                    fire_gather(g + 1, ip1, r)
            fire_store(g, tb)

        for i in range(3):
            fire_idx(i, i)
        wait_idx(0)
        for r in range(_GRP):
            fire_gather(0, 0, r)
        step(0, 0, 1, guarded=False)
        step(1, 1, 2, guarded=False)

        assert (groups_per_w - 2) % 4 == 0

        @pl.loop(0, (groups_per_w - 2) // 4)
        def _(t):
            g0 = 4 * t + 2
            step(g0, 0, 3, guarded=True)
            step(g0 + 1, 1, 0, guarded=True)
            step(g0 + 2, 0, 1, guarded=True)
            step(g0 + 3, 1, 2, guarded=True)

        wait_store(0)
        wait_store(1)

    return gather_kernel


def kernel(inputs, embedding):
    b, s = inputs.shape
    v, d = embedding.shape
    idx = inputs.T.reshape(-1, _BLK).astype(jnp.int32)
    y = _make_gather(b, s, d)(embedding, idx)
    return y.transpose(2, 4, 0, 1, 3).reshape(b, s, d)


# speedup vs baseline: 1.1524x; 1.1524x over previous
import functools

import jax
import jax.numpy as jnp
from jax import lax
from jax.experimental import pallas as pl
from jax.experimental.pallas import tpu as pltpu
from jax.experimental.pallas import tpu_sc as plsc

_NUM_CORES = 2
_NUM_SUBCORES = 16
_NUM_WORKERS = _NUM_CORES * _NUM_SUBCORES

_BLK = 128
_NBUF = 4


@functools.lru_cache(maxsize=None)
def _make_gather(n_b: int, n_s: int, d: int):
    assert d == 64 and n_b % _BLK == 0
    n_bh = n_b // _BLK
    n_blocks = n_s * n_bh
    blocks_per_w = n_blocks // _NUM_WORKERS
    assert blocks_per_w % _NBUF == 0 and blocks_per_w >= 2 * _NBUF

    mesh = plsc.VectorSubcoreMesh(
        core_axis_name="c", subcore_axis_name="s",
        num_cores=_NUM_CORES, num_subcores=_NUM_SUBCORES)

    @functools.partial(
        pl.kernel,
        out_type=jax.ShapeDtypeStruct((n_s, 8, n_bh, 8, _BLK), jnp.float32),
        mesh=mesh,
        scratch_types=[
            pltpu.VMEM((blocks_per_w, _BLK), jnp.int32),
            [pltpu.VMEM((_BLK, d), jnp.float32)] * _NBUF,
            [pltpu.VMEM((8, 8, _BLK + 1), jnp.float32)] * _NBUF,
            [pltpu.SemaphoreType.DMA] * _NBUF,
            [pltpu.SemaphoreType.DMA] * _NBUF,
        ],
        compiler_params=pltpu.CompilerParams(
            use_tc_tiling_on_sc=False, needs_layout_passes=False),
    )
    def gather_kernel(table, idx_hbm, out5, idx_v, bufs, tbufs, gsems, ssems):
        wid = lax.axis_index("s") * _NUM_CORES + lax.axis_index("c")
        blk_base = wid * blocks_per_w
        pltpu.sync_copy(idx_hbm.at[pl.ds(blk_base, blocks_per_w)], idx_v)

        d16 = lax.iota(jnp.int32, 16)
        dh_vecs = [(d16 + 16 * k) >> 3 for k in range(4)]
        dl_vecs = [(d16 + 16 * k) & 7 for k in range(4)]

        def fire_gather(i, b):
            pltpu.async_copy(table.at[idx_v.at[i]], bufs[b], gsems[b])

        def wait_gather(b):
            pltpu.make_async_copy(
                table.at[idx_v.at[0]], bufs[b], gsems[b]).wait()

        def fire_store(i, b):
            blk = blk_base + i
            s = blk >> 5
            bh = blk & (n_bh - 1)
            pltpu.async_copy(tbufs[b].at[:, :, pl.ds(0, _BLK)],
                             out5.at[s, :, bh], ssems[b])

        def wait_store(b):
            pltpu.make_async_copy(
                tbufs[b].at[:, :, pl.ds(0, _BLK)],
                out5.at[0, :, 0], ssems[b]).wait()

        def transpose_block(b):
            buf_r, tbuf_r = bufs[b], tbufs[b]

            @pl.loop(0, _BLK, unroll=8)
            def _(bl):
                blv = jnp.full((16,), bl, jnp.int32)
                vs = [buf_r[bl, pl.ds(16 * k, 16)] for k in range(4)]
                for k in range(4):
                    plsc.store_scatter(
                        tbuf_r, [dh_vecs[k], dl_vecs[k], blv], vs[k])

        def step(i, j, guard_store, guard_gather):
            if guard_store:
                wait_store(j)
            ahead = (j + 3) % _NBUF
            if guard_gather:

                @pl.when(i + 3 < blocks_per_w)
                def _():
                    fire_gather(i + 3, ahead)
            else:
                fire_gather(i + 3, ahead)
            wait_gather(j)
            transpose_block(j)
            fire_store(i, j)

        fire_gather(0, 0)
        fire_gather(1, 1)
        fire_gather(2, 2)
        for j in range(_NBUF):
            step(j, j, guard_store=False, guard_gather=False)

        @pl.loop(1, blocks_per_w // _NBUF)
        def _(t):
            for j in range(_NBUF):
                step(_NBUF * t + j, j, guard_store=True, guard_gather=True)

        for j in range(_NBUF):
            wait_store(j)

    return gather_kernel


def kernel(inputs, embedding):
    b, s = inputs.shape
    v, d = embedding.shape
    idx = inputs.T.reshape(-1, _BLK).astype(jnp.int32)
    y = _make_gather(b, s, d)(embedding, idx)
    return y.transpose(2, 4, 0, 1, 3).reshape(b, s, d)
